# Initial kernel scaffold; baseline (speedup 1.0000x reference)
import jax
import jax.numpy as jnp
from jax.experimental import pallas as pl


def kernel(node_species, embeddings):
    raise NotImplementedError("write your pallas kernel here")


# speedup vs baseline: 1.7136x; 1.7136x over previous
import functools

import jax
import jax.numpy as jnp
from jax import lax
from jax.experimental import pallas as pl
from jax.experimental.pallas import tpu as pltpu
from jax.experimental.pallas import tpu_sc as plsc

_NUM_CORES = 2
_NUM_SUBCORES = 16
_NW = _NUM_CORES * _NUM_SUBCORES
_CHUNK = 128


def _scale_body(emb_ref, w_ref):
    scale = 1.0 / jnp.sqrt(jnp.float32(emb_ref.shape[0]))
    w_ref[...] = emb_ref[...] * scale


def _make_gather(B, V, D):
    nfull = B // _CHUNK
    tail = B - nfull * _CHUNK
    nchunks = nfull + (1 if tail else 0)
    kmax = -(-nchunks // _NW)

    mesh = plsc.VectorSubcoreMesh(core_axis_name="c", subcore_axis_name="s")

    @functools.partial(
        pl.kernel,
        mesh=mesh,
        out_type=jax.ShapeDtypeStruct((B, D), jnp.float32),
        scratch_types=[
            pltpu.VMEM((_CHUNK,), jnp.int32),
            pltpu.VMEM((_CHUNK, D), jnp.float32),
            pltpu.SemaphoreType.DMA,
        ],
    )
    def gather(w_hbm, idx_hbm, out_hbm, idx_v, rows_v, sem):
        wid = lax.axis_index("s") * _NUM_CORES + lax.axis_index("c")
        if tail:
            zeros = jnp.zeros((16,), jnp.int32)
            for i in range(_CHUNK // 16):
                idx_v[pl.ds(i * 16, 16)] = zeros
        for k in range(kmax):
            cid = wid + k * _NW
            base = cid * _CHUNK

            @pl.when(cid < nfull)
            def _():
                pltpu.sync_copy(idx_hbm.at[pl.ds(base, _CHUNK)], idx_v)
                pltpu.async_copy(w_hbm.at[idx_v], rows_v, sem).wait()
                pltpu.sync_copy(rows_v, out_hbm.at[pl.ds(base, _CHUNK)])

            if tail:
                @pl.when(cid == nfull)
                def _():
                    pltpu.sync_copy(idx_hbm.at[pl.ds(base, tail)],
                                    idx_v.at[pl.ds(0, tail)])
                    pltpu.async_copy(w_hbm.at[idx_v], rows_v, sem).wait()
                    pltpu.sync_copy(rows_v.at[pl.ds(0, tail)],
                                    out_hbm.at[pl.ds(base, tail)])

    return gather


def kernel(node_species, embeddings):
    V, D = embeddings.shape
    B = node_species.shape[0]
    w = pl.pallas_call(
        _scale_body,
        out_shape=jax.ShapeDtypeStruct((V, D), jnp.float32),
    )(embeddings)
    idx = node_species.astype(jnp.int32)
    return _make_gather(B, V, D)(w, idx)


# speedup vs baseline: 3.3862x; 1.9761x over previous
import functools

import jax
import jax.numpy as jnp
from jax import lax
from jax.experimental import pallas as pl
from jax.experimental.pallas import tpu as pltpu
from jax.experimental.pallas import tpu_sc as plsc

_NUM_CORES = 2
_NUM_SUBCORES = 16
_NW = _NUM_CORES * _NUM_SUBCORES
_CHUNK = 128


def _scale_body(emb_ref, w_ref):
    scale = 1.0 / jnp.sqrt(jnp.float32(emb_ref.shape[0]))
    w_ref[...] = emb_ref[...] * scale


def _make_gather(B, V, D):
    nfull = B // _CHUNK
    tail = B - nfull * _CHUNK
    nchunks = nfull + (1 if tail else 0)
    kmax = -(-nchunks // _NW)

    mesh = plsc.VectorSubcoreMesh(core_axis_name="c", subcore_axis_name="s")

    @functools.partial(
        pl.kernel,
        mesh=mesh,
        out_type=jax.ShapeDtypeStruct((B, D), jnp.float32),
        scratch_types=[
            pltpu.VMEM((_CHUNK,), jnp.int32),
            pltpu.VMEM((_CHUNK, D), jnp.float32),
            pltpu.VMEM((V, D), jnp.float32),
            pltpu.VMEM_SHARED((V, D), jnp.float32),
            pltpu.SemaphoreType.DMA,
        ],
    )
    def gather(w_hbm, idx_hbm, out_hbm, idx_v, rows_v, tab_v, w_sp, sem):
        s = lax.axis_index("s")
        wid = s * _NUM_CORES + lax.axis_index("c")

        @pl.when(s == 0)
        def _():
            pltpu.sync_copy(w_hbm, tab_v)
            pltpu.sync_copy(tab_v, w_sp)

        plsc.subcore_barrier()
        if tail:
            zeros = jnp.zeros((16,), jnp.int32)
            for i in range(_CHUNK // 16):
                idx_v[pl.ds(i * 16, 16)] = zeros
        for k in range(kmax):
            cid = wid + k * _NW
            base = cid * _CHUNK

            @pl.when(cid < nfull)
            def _():
                pltpu.sync_copy(idx_hbm.at[pl.ds(base, _CHUNK)], idx_v)
                pltpu.async_copy(w_sp.at[idx_v], rows_v, sem).wait()
                pltpu.sync_copy(rows_v, out_hbm.at[pl.ds(base, _CHUNK)])

            if tail:
                @pl.when(cid == nfull)
                def _():
                    pltpu.sync_copy(idx_hbm.at[pl.ds(base, tail)],
                                    idx_v.at[pl.ds(0, tail)])
                    pltpu.async_copy(w_sp.at[idx_v], rows_v, sem).wait()
                    pltpu.sync_copy(rows_v.at[pl.ds(0, tail)],
                                    out_hbm.at[pl.ds(base, tail)])

    return gather


def kernel(node_species, embeddings):
    V, D = embeddings.shape
    B = node_species.shape[0]
    w = pl.pallas_call(
        _scale_body,
        out_shape=jax.ShapeDtypeStruct((V, D), jnp.float32),
    )(embeddings)
    idx = node_species.astype(jnp.int32)
    return _make_gather(B, V, D)(w, idx)


# speedup vs baseline: 5.2000x; 1.5356x over previous
import functools

import jax
import jax.numpy as jnp
from jax import lax
from jax.experimental import pallas as pl
from jax.experimental.pallas import tpu as pltpu
from jax.experimental.pallas import tpu_sc as plsc

_NUM_CORES = 2
_NUM_SUBCORES = 16
_NW = _NUM_CORES * _NUM_SUBCORES
_CHUNK = 128


def _scale_body(emb_ref, w_ref):
    scale = 1.0 / jnp.sqrt(jnp.float32(emb_ref.shape[0]))
    w_ref[...] = emb_ref[...] * scale


def _make_gather(B, V, D):
    nfull = B // _CHUNK
    tail = B - nfull * _CHUNK
    kmax = -(-nfull // _NW)

    mesh = plsc.VectorSubcoreMesh(core_axis_name="c", subcore_axis_name="s")

    @functools.partial(
        pl.kernel,
        mesh=mesh,
        out_type=jax.ShapeDtypeStruct((B, D), jnp.float32),
        scratch_types=[
            pltpu.VMEM((2, _CHUNK), jnp.int32),
            pltpu.VMEM((2, _CHUNK, D), jnp.float32),
            pltpu.VMEM((V, D), jnp.float32),
            pltpu.VMEM_SHARED((V, D), jnp.float32),
            pltpu.SemaphoreType.DMA,
            pltpu.SemaphoreType.DMA,
            pltpu.SemaphoreType.DMA,
            pltpu.SemaphoreType.DMA,
            pltpu.SemaphoreType.DMA,
        ],
    )
    def gather(w_hbm, idx_hbm, out_hbm, idx_v, rows_v, tab_v, w_sp,
               gsem, isem0, isem1, ssem0, ssem1):
        isem = (isem0, isem1)
        ssem = (ssem0, ssem1)
        s = lax.axis_index("s")
        wid = s * _NUM_CORES + lax.axis_index("c")

        @pl.when(s == 0)
        def _():
            pltpu.sync_copy(w_hbm, tab_v)
            pltpu.sync_copy(tab_v, w_sp)

        plsc.subcore_barrier()

        def cid(k):
            return wid + k * _NW

        def idx_start(k):
            if k >= kmax:
                return
            b = k % 2

            @pl.when(cid(k) < nfull)
            def _():
                pltpu.async_copy(idx_hbm.at[pl.ds(cid(k) * _CHUNK, _CHUNK)],
                                 idx_v.at[b], isem[b])

        def store_desc(k):
            b = k % 2
            return pltpu.make_async_copy(
                rows_v.at[b], out_hbm.at[pl.ds(cid(k) * _CHUNK, _CHUNK)],
                ssem[b])

        idx_start(0)
        idx_start(1)

        for k in range(kmax):
            b = k % 2
            valid = cid(k) < nfull

            if k >= 2:
                @pl.when(valid)
                def _():
                    store_desc(k - 2).wait()

            @pl.when(valid)
            def _():
                pltpu.make_async_copy(
                    idx_hbm.at[pl.ds(cid(k) * _CHUNK, _CHUNK)],
                    idx_v.at[b], isem[b]).wait()
                pltpu.async_copy(w_sp.at[idx_v.at[b]], rows_v.at[b],
                                 gsem).wait()
                store_desc(k).start()

            idx_start(k + 2)

        for k in range(max(0, kmax - 3), kmax):
            @pl.when((cid(k) < nfull) & (cid(k + 2) >= nfull))
            def _():
                store_desc(k).wait()

        if tail:
            @pl.when(wid == (nfull % _NW))
            def _():
                base = nfull * _CHUNK
                pltpu.sync_copy(idx_hbm.at[pl.ds(base, tail)],
                                idx_v.at[0, pl.ds(0, tail)])
                pltpu.async_copy(w_sp.at[idx_v.at[0]], rows_v.at[0],
                                 gsem).wait()
                pltpu.sync_copy(rows_v.at[0, pl.ds(0, tail)],
                                out_hbm.at[pl.ds(base, tail)])

    return gather


def kernel(node_species, embeddings):
    V, D = embeddings.shape
    B = node_species.shape[0]
    w = pl.pallas_call(
        _scale_body,
        out_shape=jax.ShapeDtypeStruct((V, D), jnp.float32),
    )(embeddings)
    idx = node_species.astype(jnp.int32)
    return _make_gather(B, V, D)(w, idx)


# speedup vs baseline: 5.4576x; 1.0495x over previous
import functools

import jax
import jax.numpy as jnp
import numpy as np
from jax import lax
from jax.experimental import pallas as pl
from jax.experimental.pallas import tpu as pltpu
from jax.experimental.pallas import tpu_sc as plsc

_NUM_CORES = 2
_NUM_SUBCORES = 16
_NW = _NUM_CORES * _NUM_SUBCORES
_CHUNK = 128
_LANES = 16


def _make_kernel(B, V, VP, D):
    nfull = B // _CHUNK
    tail = B - nfull * _CHUNK
    kmax = -(-nfull // _NW)
    rows_per = 8
    n_stagers = VP // rows_per
    scale = float(np.float32(1.0) / np.sqrt(np.float32(V)))

    mesh = plsc.VectorSubcoreMesh(core_axis_name="c", subcore_axis_name="s")

    @functools.partial(
        pl.kernel,
        mesh=mesh,
        out_type=jax.ShapeDtypeStruct((B, D), jnp.float32),
        scratch_types=[
            pltpu.VMEM((2, _CHUNK), jnp.int32),
            pltpu.VMEM((2, _CHUNK, D), jnp.float32),
            pltpu.VMEM((rows_per, D), jnp.float32),
            pltpu.VMEM_SHARED((VP, D), jnp.float32),
            pltpu.SemaphoreType.DMA,
            pltpu.SemaphoreType.DMA,
            pltpu.SemaphoreType.DMA,
            pltpu.SemaphoreType.DMA,
            pltpu.SemaphoreType.DMA,
        ],
    )
    def k(emb_hbm, idx_hbm, out_hbm, idx_v, rows_v, tab_v, w_sp,
          gsem, isem0, isem1, ssem0, ssem1):
        isem = (isem0, isem1)
        ssem = (ssem0, ssem1)
        s = lax.axis_index("s")
        wid = s * _NUM_CORES + lax.axis_index("c")

        @pl.when(s < n_stagers)
        def _():
            r0 = s * rows_per
            pltpu.sync_copy(emb_hbm.at[pl.ds(r0, rows_per)], tab_v)
            for dr in range(rows_per):
                for j in range(D // _LANES):
                    col = pl.ds(j * _LANES, _LANES)
                    tab_v[dr, col] = tab_v[dr, col] * scale
            pltpu.sync_copy(tab_v, w_sp.at[pl.ds(r0, rows_per)])

        plsc.subcore_barrier()

        def cid(k_):
            return wid + k_ * _NW

        def idx_start(k_):
            if k_ >= kmax:
                return
            b = k_ % 2

            @pl.when(cid(k_) < nfull)
            def _():
                pltpu.async_copy(idx_hbm.at[pl.ds(cid(k_) * _CHUNK, _CHUNK)],
                                 idx_v.at[b], isem[b])

        def store_desc(k_):
            b = k_ % 2
            return pltpu.make_async_copy(
                rows_v.at[b], out_hbm.at[pl.ds(cid(k_) * _CHUNK, _CHUNK)],
                ssem[b])

        idx_start(0)
        idx_start(1)

        for k_ in range(kmax):
            b = k_ % 2
            valid = cid(k_) < nfull

            if k_ >= 2:
                @pl.when(valid)
                def _():
                    store_desc(k_ - 2).wait()

            @pl.when(valid)
            def _():
                pltpu.make_async_copy(
                    idx_hbm.at[pl.ds(cid(k_) * _CHUNK, _CHUNK)],
                    idx_v.at[b], isem[b]).wait()
                pltpu.async_copy(w_sp.at[idx_v.at[b]], rows_v.at[b],
                                 gsem).wait()
                store_desc(k_).start()

            idx_start(k_ + 2)

        for k_ in range(max(0, kmax - 3), kmax):
            @pl.when((cid(k_) < nfull) & (cid(k_ + 2) >= nfull))
            def _():
                store_desc(k_).wait()

        if tail:
            @pl.when(wid == (nfull % _NW))
            def _():
                base = nfull * _CHUNK
                pltpu.sync_copy(idx_hbm.at[pl.ds(base, tail)],
                                idx_v.at[0, pl.ds(0, tail)])
                pltpu.async_copy(w_sp.at[idx_v.at[0]], rows_v.at[0],
                                 gsem).wait()
                pltpu.sync_copy(rows_v.at[0, pl.ds(0, tail)],
                                out_hbm.at[pl.ds(base, tail)])

    return k


def kernel(node_species, embeddings):
    V, D = embeddings.shape
    B = node_species.shape[0]
    VP = -(-V // _NW) * _NW
    emb_p = jnp.pad(embeddings, ((0, VP - V), (0, 0)))
    idx = node_species.astype(jnp.int32)
    return _make_kernel(B, V, VP, D)(emb_p, idx)


# speedup vs baseline: 5.5428x; 1.0156x over previous
import functools

import jax
import jax.numpy as jnp
import numpy as np
from jax import lax
from jax.experimental import pallas as pl
from jax.experimental.pallas import tpu as pltpu
from jax.experimental.pallas import tpu_sc as plsc

_NUM_CORES = 2
_NUM_SUBCORES = 16
_NW = _NUM_CORES * _NUM_SUBCORES
_CHUNK = 128
_LANES = 16


def _make_kernel(B, V, D):
    nfull = B // _CHUNK
    tail = B - nfull * _CHUNK
    kmax = -(-nfull // _NW)
    n_stage8 = V // 8
    v_rem = V - n_stage8 * 8
    scale = float(np.float32(1.0) / np.sqrt(np.float32(V)))

    mesh = plsc.VectorSubcoreMesh(core_axis_name="c", subcore_axis_name="s")

    @functools.partial(
        pl.kernel,
        mesh=mesh,
        out_type=jax.ShapeDtypeStruct((B, D), jnp.float32),
        scratch_types=[
            pltpu.VMEM((2, _CHUNK), jnp.int32),
            pltpu.VMEM((2, _CHUNK, D), jnp.float32),
            pltpu.VMEM((8, D), jnp.float32),
            pltpu.VMEM_SHARED((V, D), jnp.float32),
            pltpu.SemaphoreType.DMA,
            pltpu.SemaphoreType.DMA,
            pltpu.SemaphoreType.DMA,
            pltpu.SemaphoreType.DMA,
            pltpu.SemaphoreType.DMA,
        ],
    )
    def k(emb_hbm, idx_hbm, out_hbm, idx_v, rows_v, tab_v, w_sp,
          gsem, isem0, isem1, ssem0, ssem1):
        isem = (isem0, isem1)
        ssem = (ssem0, ssem1)
        s = lax.axis_index("s")
        wid = s * _NUM_CORES + lax.axis_index("c")

        def stage(r0, nr):
            pltpu.sync_copy(emb_hbm.at[pl.ds(r0, nr)], tab_v.at[pl.ds(0, nr)])
            for dr in range(nr):
                for j in range(D // _LANES):
                    col = pl.ds(j * _LANES, _LANES)
                    tab_v[dr, col] = tab_v[dr, col] * scale
            pltpu.sync_copy(tab_v.at[pl.ds(0, nr)], w_sp.at[pl.ds(r0, nr)])

        @pl.when(s < n_stage8)
        def _():
            stage(s * 8, 8)

        if v_rem:
            @pl.when(s == n_stage8)
            def _():
                stage(n_stage8 * 8, v_rem)

        plsc.subcore_barrier()

        def cid(k_):
            return wid + k_ * _NW

        def idx_start_b(k_, b):
            @pl.when(cid(k_) < nfull)
            def _():
                pltpu.async_copy(idx_hbm.at[pl.ds(cid(k_) * _CHUNK, _CHUNK)],
                                 idx_v.at[b], isem[b])

        def store_desc_b(k_, b):
            return pltpu.make_async_copy(
                rows_v.at[b], out_hbm.at[pl.ds(cid(k_) * _CHUNK, _CHUNK)],
                ssem[b])

        def process(k_, b, drain):
            valid = cid(k_) < nfull

            if drain:
                @pl.when(valid)
                def _():
                    store_desc_b(k_ - 2, b).wait()

            @pl.when(valid)
            def _():
                pltpu.make_async_copy(
                    idx_hbm.at[pl.ds(cid(k_) * _CHUNK, _CHUNK)],
                    idx_v.at[b], isem[b]).wait()
                pltpu.async_copy(w_sp.at[idx_v.at[b]], rows_v.at[b],
                                 gsem).wait()
                store_desc_b(k_, b).start()

            idx_start_b(k_ + 2, b)

        idx_start_b(0, 0)
        idx_start_b(1, 1)
        process(0, 0, drain=False)
        process(1, 1, drain=False)

        npairs = (kmax - 2) // 2

        @pl.loop(1, 1 + npairs)
        def _(t):
            process(2 * t, 0, drain=True)
            process(2 * t + 1, 1, drain=True)

        for k_ in range(2 + 2 * npairs, kmax):
            process(k_, k_ % 2, drain=True)

        for k_ in range(max(0, kmax - 3), kmax):
            @pl.when((cid(k_) < nfull) & (cid(k_ + 2) >= nfull))
            def _():
                store_desc_b(k_, k_ % 2).wait()

        if tail:
            @pl.when(wid == (nfull % _NW))
            def _():
                base = nfull * _CHUNK
                pltpu.sync_copy(idx_hbm.at[pl.ds(base, tail)],
                                idx_v.at[0, pl.ds(0, tail)])
                pltpu.async_copy(w_sp.at[idx_v.at[0]], rows_v.at[0],
                                 gsem).wait()
                pltpu.sync_copy(rows_v.at[0, pl.ds(0, tail)],
                                out_hbm.at[pl.ds(base, tail)])

    return k


def kernel(node_species, embeddings):
    V, D = embeddings.shape
    B = node_species.shape[0]
    idx = node_species.astype(jnp.int32)
    return _make_kernel(B, V, D)(embeddings, idx)
